# baseline (speedup 1.0000x reference)
import jax
import jax.numpy as jnp
from jax import lax
from jax.experimental import pallas as pl
from jax.experimental.pallas import tpu as pltpu
from jax.experimental.pallas import tpu_sc as plsc

N = 10000
M = 20000
NNZ = 320000
DIN = 128
DOUT = 32

NC = 2
NS = 16
NW = NC * NS
L = 16

IDXW = 128
RPW = 80
NB = 8
RPW0 = 48
RPW1 = 2 * RPW - RPW0
ROWS = NW * RPW
NNZ_PAD = ROWS * IDXW
PAD = NNZ_PAD - NNZ

M_PAD = 20480
M_SLAB = M_PAD // NS
N_PAD = 10240
N_SLAB = N_PAD // NS
ZROWS = 320

_MESH = plsc.VectorSubcoreMesh(core_axis_name="c", subcore_axis_name="s")
_SC_PARAMS = pltpu.CompilerParams(use_tc_tiling_on_sc=False)


def _splat(vec, j):
    idx = jnp.full((L, 1), j, dtype=jnp.int32)
    dn = lax.GatherDimensionNumbers(
        offset_dims=(), collapsed_slice_dims=(0,), start_index_map=(0,))
    return lax.gather(vec, idx, dn, (1,),
                      mode=lax.GatherScatterMode.PROMISE_IN_BOUNDS)


def _stage(src2d, dst, cid, sid):
    @pl.when(cid == 0)
    def _():
        pltpu.sync_copy(src2d.at[pl.ds(sid * RPW0, RPW0)],
                        dst.at[pl.ds(0, RPW0)])

    @pl.when(cid == 1)
    def _():
        pltpu.sync_copy(src2d.at[pl.ds(NS * RPW0 + sid * RPW1, RPW1)],
                        dst.at[pl.ds(0, RPW1)])


def _zero_rows(zbuf, nrows):
    z = jnp.zeros((L,), jnp.float32)

    def body(i, _):
        zbuf[i, 0:16] = z
        zbuf[i, 16:32] = z
        return 0

    lax.fori_loop(0, nrows, body, 0, unroll=False)


def _sc_a_body(xw1, v2d, e2d, a2d, xe_out,
               vidx, eidx, attv, rows, zbuf, xe_acc, gsem, ssem):
    cid = lax.axis_index("c")
    sid = lax.axis_index("s")

    _zero_rows(zbuf, ZROWS)
    for k in range(M_SLAB // ZROWS):
        pltpu.sync_copy(zbuf, xe_acc.at[pl.ds(sid * M_SLAB + k * ZROWS,
                                              ZROWS)])
    plsc.subcore_barrier()

    _stage(v2d, vidx, cid, sid)
    _stage(e2d, eidx, cid, sid)
    _stage(a2d, attv, cid, sid)
    nsteps = jnp.where(cid == 0, RPW0 // NB, RPW1 // NB)

    def scale_rows(j, b):
        def blk_body(blk, _):
            ab = attv[j, pl.ds(blk * L, L)]
            for r in range(L):
                s = _splat(ab, r)
                rr = blk * L + r
                rows[b, rr, 0:16] = rows[b, rr, 0:16] * s
                rows[b, rr, 16:32] = rows[b, rr, 16:32] * s
            return 0

        lax.fori_loop(0, IDXW // L, blk_body, 0, unroll=False)

    def body(step, _):
        g = []
        for b in range(NB):
            j = step * NB + b
            g.append(pltpu.async_copy(xw1.at[vidx.at[j]], rows.at[b], gsem))
        sc = []
        for b in range(NB):
            j = step * NB + b
            g[b].wait()
            scale_rows(j, b)
            sc.append(pltpu.async_copy(
                rows.at[b], xe_acc.at[eidx.at[j]], ssem, add=True))
        for d in sc:
            d.wait()
        return 0

    lax.fori_loop(0, nsteps, body, 0, unroll=False)
    plsc.subcore_barrier()

    pltpu.sync_copy(xe_acc.at[pl.ds(sid * M_SLAB, M_SLAB)],
                    xe_out.at[cid, pl.ds(sid * M_SLAB, M_SLAB)])


_sc_a = pl.kernel(
    _sc_a_body,
    out_type=jax.ShapeDtypeStruct((NC, M_PAD, DOUT), jnp.float32),
    mesh=_MESH,
    scratch_types=[
        pltpu.VMEM((RPW1, IDXW), jnp.int32),
        pltpu.VMEM((RPW1, IDXW), jnp.int32),
        pltpu.VMEM((RPW1, IDXW), jnp.float32),
        pltpu.VMEM((NB, IDXW, DOUT), jnp.float32),
        pltpu.VMEM((ZROWS, DOUT), jnp.float32),
        pltpu.VMEM_SHARED((M_PAD, DOUT), jnp.float32),
        pltpu.SemaphoreType.DMA,
        pltpu.SemaphoreType.DMA,
    ],
    compiler_params=_SC_PARAMS,
)


def _sc_b_body(y, v2d, e2d, s_out, cnt_out,
               vidx, eidx, rows, ones, zbuf, zcnt, s_acc, cnt_acc,
               gsem, ssem, csem):
    cid = lax.axis_index("c")
    sid = lax.axis_index("s")

    _zero_rows(zbuf, ZROWS)
    one = jnp.full((L,), 1.0, jnp.float32)
    zv = jnp.zeros((L,), jnp.float32)
    for i in range(IDXW // L):
        ones[pl.ds(i * L, L)] = one
    for i in range(N_SLAB // L):
        zcnt[pl.ds(i * L, L)] = zv
    for k in range(N_SLAB // ZROWS):
        pltpu.sync_copy(zbuf, s_acc.at[pl.ds(sid * N_SLAB + k * ZROWS,
                                             ZROWS)])
    pltpu.sync_copy(zcnt, cnt_acc.at[pl.ds(sid * N_SLAB, N_SLAB)])
    plsc.subcore_barrier()

    _stage(v2d, vidx, cid, sid)
    _stage(e2d, eidx, cid, sid)
    nsteps = jnp.where(cid == 0, RPW0 // NB, RPW1 // NB)

    def body(step, _):
        g = []
        for b in range(NB):
            j = step * NB + b
            g.append(pltpu.async_copy(y.at[eidx.at[j]], rows.at[b], gsem))
        sc = []
        for b in range(NB):
            j = step * NB + b
            g[b].wait()
            sc.append(pltpu.async_copy(
                rows.at[b], s_acc.at[vidx.at[j]], ssem, add=True))
            sc.append(pltpu.async_copy(
                ones, cnt_acc.at[vidx.at[j]], csem, add=True))
        for d in sc:
            d.wait()
        return 0

    lax.fori_loop(0, nsteps, body, 0, unroll=False)
    plsc.subcore_barrier()

    pltpu.sync_copy(s_acc.at[pl.ds(sid * N_SLAB, N_SLAB)],
                    s_out.at[cid, pl.ds(sid * N_SLAB, N_SLAB)])
    pltpu.sync_copy(cnt_acc.at[pl.ds(sid * N_SLAB, N_SLAB)],
                    cnt_out.at[cid, pl.ds(sid * N_SLAB, N_SLAB)])


_sc_b = pl.kernel(
    _sc_b_body,
    out_type=(
        jax.ShapeDtypeStruct((NC, N_PAD, DOUT), jnp.float32),
        jax.ShapeDtypeStruct((NC, N_PAD), jnp.float32),
    ),
    mesh=_MESH,
    scratch_types=[
        pltpu.VMEM((RPW1, IDXW), jnp.int32),
        pltpu.VMEM((RPW1, IDXW), jnp.int32),
        pltpu.VMEM((NB, IDXW, DOUT), jnp.float32),
        pltpu.VMEM((IDXW,), jnp.float32),
        pltpu.VMEM((ZROWS, DOUT), jnp.float32),
        pltpu.VMEM((N_SLAB,), jnp.float32),
        pltpu.VMEM_SHARED((N_PAD, DOUT), jnp.float32),
        pltpu.VMEM_SHARED((N_PAD,), jnp.float32),
        pltpu.SemaphoreType.DMA,
        pltpu.SemaphoreType.DMA,
        pltpu.SemaphoreType.DMA,
    ],
    compiler_params=_SC_PARAMS,
)


def _tc1_body(x_ref, w1t_ref, b1_ref, w2t_ref, b2_ref, xw1_ref, d_ref):
    x = x_ref[...]
    xw1_ref[...] = (
        jnp.dot(x, w1t_ref[...], preferred_element_type=jnp.float32)
        + b1_ref[...])
    d_ref[...] = (
        jnp.dot(x, w2t_ref[...], preferred_element_type=jnp.float32)
        + b2_ref[...])


_tc1 = pl.pallas_call(
    _tc1_body,
    out_shape=(
        jax.ShapeDtypeStruct((N, DOUT), jnp.float32),
        jax.ShapeDtypeStruct((N, DOUT), jnp.float32),
    ),
)


def _tc2_body(p0_ref, p1_ref, w2bt_ref, y_ref):
    xe = p0_ref[...] + p1_ref[...]
    y_ref[...] = jnp.dot(xe, w2bt_ref[...],
                         preferred_element_type=jnp.float32)


_tc2 = pl.pallas_call(
    _tc2_body,
    out_shape=jax.ShapeDtypeStruct((M_PAD, DOUT), jnp.float32),
)


def _tc3_body(d_ref, s0_ref, s1_ref, c_ref, x0_ref, wt_ref, wb_ref, out_ref):
    cnt = (c_ref[0, :] + c_ref[1, :])[:, None]
    xv = cnt * d_ref[...] + s0_ref[...] + s1_ref[...]
    xn = 0.5 * xv + 0.5 * x0_ref[...]
    out_ref[...] = (
        jnp.dot(xn, wt_ref[...], preferred_element_type=jnp.float32)
        + wb_ref[...])


_tc3 = pl.pallas_call(
    _tc3_body,
    out_shape=jax.ShapeDtypeStruct((N, DOUT), jnp.float32),
)


def kernel(X, vertex, edges, atts, X0, W1_w, W1_b, W2_w, W2_b, W_w, W_b):
    vg2d = jnp.concatenate(
        [vertex, jnp.zeros((PAD,), jnp.int32)]).reshape(ROWS, IDXW)
    vs2d = jnp.concatenate(
        [vertex, jnp.full((PAD,), N, jnp.int32)]).reshape(ROWS, IDXW)
    e2d = jnp.concatenate(
        [edges, jnp.full((PAD,), M, jnp.int32)]).reshape(ROWS, IDXW)
    a2d = jnp.concatenate(
        [atts.reshape(NNZ), jnp.zeros((PAD,), jnp.float32)]).reshape(ROWS, IDXW)

    w1t = W1_w.T
    w2at = W2_w[:, :DIN].T
    w2bt = W2_w[:, DIN:].T
    wt = W_w.T
    b1 = W1_b.reshape(1, DOUT)
    b2 = W2_b.reshape(1, DOUT)
    wb = W_b.reshape(1, DOUT)

    xw1, d = _tc1(X, w1t, b1, w2at, b2)
    xe_parts = _sc_a(xw1, vg2d, e2d, a2d)
    y = _tc2(xe_parts[0], xe_parts[1], w2bt)
    s_parts, cnt_parts = _sc_b(y, vs2d, e2d)
    out = _tc3(d, s_parts[0, :N], s_parts[1, :N], cnt_parts[:, :N],
               X0, wt, wb)
    return out


# speedup vs baseline: 1.1516x; 1.1516x over previous
import jax
import jax.numpy as jnp
from jax import lax
from jax.experimental import pallas as pl
from jax.experimental.pallas import tpu as pltpu
from jax.experimental.pallas import tpu_sc as plsc

N = 10000
M = 20000
NNZ = 320000
DIN = 128
DOUT = 32

NC = 2
NS = 16
NW = NC * NS
L = 16

IDXW = 128
RPW = 80
NB = 8
RPW0 = 112
RPW1 = 2 * RPW - RPW0
RPW_MAX = max(RPW0, RPW1)
ROWS = NW * RPW
NNZ_PAD = ROWS * IDXW
PAD = NNZ_PAD - NNZ

M_PAD = 20480
M_SLAB = M_PAD // NS
N_PAD = 10240
N_SLAB = N_PAD // NS
ZROWS = 320

_MESH = plsc.VectorSubcoreMesh(core_axis_name="c", subcore_axis_name="s")
_SC_PARAMS = pltpu.CompilerParams(use_tc_tiling_on_sc=False)


def _splat(vec, j):
    idx = jnp.full((L, 1), j, dtype=jnp.int32)
    dn = lax.GatherDimensionNumbers(
        offset_dims=(), collapsed_slice_dims=(0,), start_index_map=(0,))
    return lax.gather(vec, idx, dn, (1,),
                      mode=lax.GatherScatterMode.PROMISE_IN_BOUNDS)


def _stage(src2d, dst, cid, sid):
    @pl.when(cid == 0)
    def _():
        pltpu.sync_copy(src2d.at[pl.ds(sid * RPW0, RPW0)],
                        dst.at[pl.ds(0, RPW0)])

    @pl.when(cid == 1)
    def _():
        pltpu.sync_copy(src2d.at[pl.ds(NS * RPW0 + sid * RPW1, RPW1)],
                        dst.at[pl.ds(0, RPW1)])


def _zero_rows(zbuf, nrows):
    z = jnp.zeros((L,), jnp.float32)

    def body(i, _):
        zbuf[i, 0:16] = z
        zbuf[i, 16:32] = z
        return 0

    lax.fori_loop(0, nrows, body, 0, unroll=False)


def _sc_a_body(xw1, v2d, e2d, a2d, xe_out,
               vidx, eidx, attv, rows, zbuf, xe_acc, gsem, ssem):
    cid = lax.axis_index("c")
    sid = lax.axis_index("s")

    _zero_rows(zbuf, ZROWS)
    for k in range(M_SLAB // ZROWS):
        pltpu.sync_copy(zbuf, xe_acc.at[pl.ds(sid * M_SLAB + k * ZROWS,
                                              ZROWS)])
    plsc.subcore_barrier()

    _stage(v2d, vidx, cid, sid)
    _stage(e2d, eidx, cid, sid)
    _stage(a2d, attv, cid, sid)
    nsteps = jnp.where(cid == 0, RPW0 // NB, RPW1 // NB)

    def scale_rows(j, b):
        def blk_body(blk, _):
            ab = attv[j, pl.ds(blk * L, L)]
            for r in range(L):
                s = _splat(ab, r)
                rr = blk * L + r
                rows[b, rr, 0:16] = rows[b, rr, 0:16] * s
                rows[b, rr, 16:32] = rows[b, rr, 16:32] * s
            return 0

        lax.fori_loop(0, IDXW // L, blk_body, 0, unroll=False)

    def body(step, _):
        g = []
        for b in range(NB):
            j = step * NB + b
            g.append(pltpu.async_copy(xw1.at[vidx.at[j]], rows.at[b], gsem))
        sc = []
        for b in range(NB):
            j = step * NB + b
            g[b].wait()
            scale_rows(j, b)
            sc.append(pltpu.async_copy(
                rows.at[b], xe_acc.at[eidx.at[j]], ssem, add=True))
        for d in sc:
            d.wait()
        return 0

    lax.fori_loop(0, nsteps, body, 0, unroll=False)
    plsc.subcore_barrier()

    pltpu.sync_copy(xe_acc.at[pl.ds(sid * M_SLAB, M_SLAB)],
                    xe_out.at[cid, pl.ds(sid * M_SLAB, M_SLAB)])


_sc_a = pl.kernel(
    _sc_a_body,
    out_type=jax.ShapeDtypeStruct((NC, M_PAD, DOUT), jnp.float32),
    mesh=_MESH,
    scratch_types=[
        pltpu.VMEM((RPW_MAX, IDXW), jnp.int32),
        pltpu.VMEM((RPW_MAX, IDXW), jnp.int32),
        pltpu.VMEM((RPW_MAX, IDXW), jnp.float32),
        pltpu.VMEM((NB, IDXW, DOUT), jnp.float32),
        pltpu.VMEM((ZROWS, DOUT), jnp.float32),
        pltpu.VMEM_SHARED((M_PAD, DOUT), jnp.float32),
        pltpu.SemaphoreType.DMA,
        pltpu.SemaphoreType.DMA,
    ],
    compiler_params=_SC_PARAMS,
)


def _sc_b_body(y, v2d, e2d, s_out, cnt_out,
               vidx, eidx, rows, ones, zbuf, zcnt, s_acc, cnt_acc,
               gsem, ssem, csem):
    cid = lax.axis_index("c")
    sid = lax.axis_index("s")

    _zero_rows(zbuf, ZROWS)
    one = jnp.full((L,), 1.0, jnp.float32)
    zv = jnp.zeros((L,), jnp.float32)
    for i in range(IDXW // L):
        ones[pl.ds(i * L, L)] = one
    for i in range(N_SLAB // L):
        zcnt[pl.ds(i * L, L)] = zv
    for k in range(N_SLAB // ZROWS):
        pltpu.sync_copy(zbuf, s_acc.at[pl.ds(sid * N_SLAB + k * ZROWS,
                                             ZROWS)])
    pltpu.sync_copy(zcnt, cnt_acc.at[pl.ds(sid * N_SLAB, N_SLAB)])
    plsc.subcore_barrier()

    _stage(v2d, vidx, cid, sid)
    _stage(e2d, eidx, cid, sid)
    nsteps = jnp.where(cid == 0, RPW0 // NB, RPW1 // NB)

    def body(step, _):
        g = []
        for b in range(NB):
            j = step * NB + b
            g.append(pltpu.async_copy(y.at[eidx.at[j]], rows.at[b], gsem))
        sc = []
        for b in range(NB):
            j = step * NB + b
            g[b].wait()
            sc.append(pltpu.async_copy(
                rows.at[b], s_acc.at[vidx.at[j]], ssem, add=True))
            sc.append(pltpu.async_copy(
                ones, cnt_acc.at[vidx.at[j]], csem, add=True))
        for d in sc:
            d.wait()
        return 0

    lax.fori_loop(0, nsteps, body, 0, unroll=False)
    plsc.subcore_barrier()

    pltpu.sync_copy(s_acc.at[pl.ds(sid * N_SLAB, N_SLAB)],
                    s_out.at[cid, pl.ds(sid * N_SLAB, N_SLAB)])
    pltpu.sync_copy(cnt_acc.at[pl.ds(sid * N_SLAB, N_SLAB)],
                    cnt_out.at[cid, pl.ds(sid * N_SLAB, N_SLAB)])


_sc_b = pl.kernel(
    _sc_b_body,
    out_type=(
        jax.ShapeDtypeStruct((NC, N_PAD, DOUT), jnp.float32),
        jax.ShapeDtypeStruct((NC, N_PAD), jnp.float32),
    ),
    mesh=_MESH,
    scratch_types=[
        pltpu.VMEM((RPW_MAX, IDXW), jnp.int32),
        pltpu.VMEM((RPW_MAX, IDXW), jnp.int32),
        pltpu.VMEM((NB, IDXW, DOUT), jnp.float32),
        pltpu.VMEM((IDXW,), jnp.float32),
        pltpu.VMEM((ZROWS, DOUT), jnp.float32),
        pltpu.VMEM((N_SLAB,), jnp.float32),
        pltpu.VMEM_SHARED((N_PAD, DOUT), jnp.float32),
        pltpu.VMEM_SHARED((N_PAD,), jnp.float32),
        pltpu.SemaphoreType.DMA,
        pltpu.SemaphoreType.DMA,
        pltpu.SemaphoreType.DMA,
    ],
    compiler_params=_SC_PARAMS,
)


def _tc1_body(x_ref, w1t_ref, b1_ref, w2t_ref, b2_ref, xw1_ref, d_ref):
    x = x_ref[...]
    xw1_ref[...] = (
        jnp.dot(x, w1t_ref[...], preferred_element_type=jnp.float32)
        + b1_ref[...])
    d_ref[...] = (
        jnp.dot(x, w2t_ref[...], preferred_element_type=jnp.float32)
        + b2_ref[...])


_tc1 = pl.pallas_call(
    _tc1_body,
    out_shape=(
        jax.ShapeDtypeStruct((N, DOUT), jnp.float32),
        jax.ShapeDtypeStruct((N, DOUT), jnp.float32),
    ),
)


def _tc2_body(p0_ref, p1_ref, w2bt_ref, y_ref):
    xe = p0_ref[...] + p1_ref[...]
    y_ref[...] = jnp.dot(xe, w2bt_ref[...],
                         preferred_element_type=jnp.float32)


_tc2 = pl.pallas_call(
    _tc2_body,
    out_shape=jax.ShapeDtypeStruct((M_PAD, DOUT), jnp.float32),
)


def _tc3_body(d_ref, s0_ref, s1_ref, c_ref, x0_ref, wt_ref, wb_ref, out_ref):
    cnt = (c_ref[0, :] + c_ref[1, :])[:, None]
    xv = cnt * d_ref[...] + s0_ref[...] + s1_ref[...]
    xn = 0.5 * xv + 0.5 * x0_ref[...]
    out_ref[...] = (
        jnp.dot(xn, wt_ref[...], preferred_element_type=jnp.float32)
        + wb_ref[...])


_tc3 = pl.pallas_call(
    _tc3_body,
    out_shape=jax.ShapeDtypeStruct((N, DOUT), jnp.float32),
)


def kernel(X, vertex, edges, atts, X0, W1_w, W1_b, W2_w, W2_b, W_w, W_b):
    vg2d = jnp.concatenate(
        [vertex, jnp.zeros((PAD,), jnp.int32)]).reshape(ROWS, IDXW)
    vs2d = jnp.concatenate(
        [vertex, jnp.full((PAD,), N, jnp.int32)]).reshape(ROWS, IDXW)
    e2d = jnp.concatenate(
        [edges, jnp.full((PAD,), M, jnp.int32)]).reshape(ROWS, IDXW)
    a2d = jnp.concatenate(
        [atts.reshape(NNZ), jnp.zeros((PAD,), jnp.float32)]).reshape(ROWS, IDXW)

    w1t = W1_w.T
    w2at = W2_w[:, :DIN].T
    w2bt = W2_w[:, DIN:].T
    wt = W_w.T
    b1 = W1_b.reshape(1, DOUT)
    b2 = W2_b.reshape(1, DOUT)
    wb = W_b.reshape(1, DOUT)

    xw1, d = _tc1(X, w1t, b1, w2at, b2)
    xe_parts = _sc_a(xw1, vg2d, e2d, a2d)
    y = _tc2(xe_parts[0], xe_parts[1], w2bt)
    s_parts, cnt_parts = _sc_b(y, vs2d, e2d)
    out = _tc3(d, s_parts[0, :N], s_parts[1, :N], cnt_parts[:, :N],
               X0, wt, wb)
    return out


# speedup vs baseline: 1.1520x; 1.0003x over previous
import jax
import jax.numpy as jnp
from jax import lax
from jax.experimental import pallas as pl
from jax.experimental.pallas import tpu as pltpu
from jax.experimental.pallas import tpu_sc as plsc

N = 10000
M = 20000
NNZ = 320000
DIN = 128
DOUT = 32

NC = 2
NS = 16
NW = NC * NS
L = 16

IDXW = 128
RPW = 80
NB = 8
RPW0 = 112
RPW1 = 2 * RPW - RPW0
RPW_MAX = max(RPW0, RPW1)
ROWS = NW * RPW
NNZ_PAD = ROWS * IDXW
PAD = NNZ_PAD - NNZ

M_PAD = 20480
M_SLAB = M_PAD // NS
N_PAD = 10240
N_SLAB = N_PAD // NS
ZROWS = 320

_MESH = plsc.VectorSubcoreMesh(core_axis_name="c", subcore_axis_name="s")
_SC_PARAMS = pltpu.CompilerParams(use_tc_tiling_on_sc=False)


def _splat(vec, j):
    idx = jnp.full((L, 1), j, dtype=jnp.int32)
    dn = lax.GatherDimensionNumbers(
        offset_dims=(), collapsed_slice_dims=(0,), start_index_map=(0,))
    return lax.gather(vec, idx, dn, (1,),
                      mode=lax.GatherScatterMode.PROMISE_IN_BOUNDS)


def _stage(src2d, dst, cid, sid):
    @pl.when(cid == 0)
    def _():
        pltpu.sync_copy(src2d.at[pl.ds(sid * RPW0, RPW0)],
                        dst.at[pl.ds(0, RPW0)])

    @pl.when(cid == 1)
    def _():
        pltpu.sync_copy(src2d.at[pl.ds(NS * RPW0 + sid * RPW1, RPW1)],
                        dst.at[pl.ds(0, RPW1)])


def _zero_rows(zbuf, nrows):
    z = jnp.zeros((L,), jnp.float32)

    def body(i, _):
        zbuf[i, 0:16] = z
        zbuf[i, 16:32] = z
        return 0

    lax.fori_loop(0, nrows, body, 0, unroll=False)


def _sc_a_body(xw1, v2d, e2d, a2d, xe_out,
               vidx, eidx, attv, rows, zbuf, xe_acc, gsem, ssem):
    cid = lax.axis_index("c")
    sid = lax.axis_index("s")

    with jax.named_scope("zero_acc"):
        _zero_rows(zbuf, ZROWS)
        for k in range(M_SLAB // ZROWS):
            pltpu.sync_copy(zbuf, xe_acc.at[pl.ds(sid * M_SLAB + k * ZROWS,
                                                  ZROWS)])
        plsc.subcore_barrier()

    with jax.named_scope("stage_idx"):
        _stage(v2d, vidx, cid, sid)
        _stage(e2d, eidx, cid, sid)
        _stage(a2d, attv, cid, sid)
    nsteps = jnp.where(cid == 0, RPW0 // NB, RPW1 // NB)

    def scale_rows(j, b):
        def blk_body(blk, _):
            ab = attv[j, pl.ds(blk * L, L)]
            for r in range(L):
                s = _splat(ab, r)
                rr = blk * L + r
                rows[b, rr, 0:16] = rows[b, rr, 0:16] * s
                rows[b, rr, 16:32] = rows[b, rr, 16:32] * s
            return 0

        lax.fori_loop(0, IDXW // L, blk_body, 0, unroll=False)

    def body(step, _):
        g = []
        for b in range(NB):
            j = step * NB + b
            g.append(pltpu.async_copy(xw1.at[vidx.at[j]], rows.at[b], gsem))
        sc = []
        for b in range(NB):
            j = step * NB + b
            g[b].wait()
            scale_rows(j, b)
            sc.append(pltpu.async_copy(
                rows.at[b], xe_acc.at[eidx.at[j]], ssem, add=True))
        for d in sc:
            d.wait()
        return 0

    with jax.named_scope("gather_scatter"):
        lax.fori_loop(0, nsteps, body, 0, unroll=False)
        plsc.subcore_barrier()

    with jax.named_scope("writeback"):
        pltpu.sync_copy(xe_acc.at[pl.ds(sid * M_SLAB, M_SLAB)],
                        xe_out.at[cid, pl.ds(sid * M_SLAB, M_SLAB)])


_sc_a = pl.kernel(
    _sc_a_body,
    out_type=jax.ShapeDtypeStruct((NC, M_PAD, DOUT), jnp.float32),
    mesh=_MESH,
    scratch_types=[
        pltpu.VMEM((RPW_MAX, IDXW), jnp.int32),
        pltpu.VMEM((RPW_MAX, IDXW), jnp.int32),
        pltpu.VMEM((RPW_MAX, IDXW), jnp.float32),
        pltpu.VMEM((NB, IDXW, DOUT), jnp.float32),
        pltpu.VMEM((ZROWS, DOUT), jnp.float32),
        pltpu.VMEM_SHARED((M_PAD, DOUT), jnp.float32),
        pltpu.SemaphoreType.DMA,
        pltpu.SemaphoreType.DMA,
    ],
    compiler_params=_SC_PARAMS,
)


def _sc_b_body(y, v2d, e2d, s_out, cnt_out,
               vidx, eidx, rows, ones, zbuf, zcnt, s_acc, cnt_acc,
               gsem, ssem, csem):
    cid = lax.axis_index("c")
    sid = lax.axis_index("s")

    _zero_rows(zbuf, ZROWS)
    one = jnp.full((L,), 1.0, jnp.float32)
    zv = jnp.zeros((L,), jnp.float32)
    for i in range(IDXW // L):
        ones[pl.ds(i * L, L)] = one
    for i in range(N_SLAB // L):
        zcnt[pl.ds(i * L, L)] = zv
    for k in range(N_SLAB // ZROWS):
        pltpu.sync_copy(zbuf, s_acc.at[pl.ds(sid * N_SLAB + k * ZROWS,
                                             ZROWS)])
    pltpu.sync_copy(zcnt, cnt_acc.at[pl.ds(sid * N_SLAB, N_SLAB)])
    plsc.subcore_barrier()

    _stage(v2d, vidx, cid, sid)
    _stage(e2d, eidx, cid, sid)
    nsteps = jnp.where(cid == 0, RPW0 // NB, RPW1 // NB)

    def body(step, _):
        g = []
        for b in range(NB):
            j = step * NB + b
            g.append(pltpu.async_copy(y.at[eidx.at[j]], rows.at[b], gsem))
        sc = []
        for b in range(NB):
            j = step * NB + b
            g[b].wait()
            sc.append(pltpu.async_copy(
                rows.at[b], s_acc.at[vidx.at[j]], ssem, add=True))
            sc.append(pltpu.async_copy(
                ones, cnt_acc.at[vidx.at[j]], csem, add=True))
        for d in sc:
            d.wait()
        return 0

    lax.fori_loop(0, nsteps, body, 0, unroll=False)
    plsc.subcore_barrier()

    pltpu.sync_copy(s_acc.at[pl.ds(sid * N_SLAB, N_SLAB)],
                    s_out.at[cid, pl.ds(sid * N_SLAB, N_SLAB)])
    pltpu.sync_copy(cnt_acc.at[pl.ds(sid * N_SLAB, N_SLAB)],
                    cnt_out.at[cid, pl.ds(sid * N_SLAB, N_SLAB)])


_sc_b = pl.kernel(
    _sc_b_body,
    out_type=(
        jax.ShapeDtypeStruct((NC, N_PAD, DOUT), jnp.float32),
        jax.ShapeDtypeStruct((NC, N_PAD), jnp.float32),
    ),
    mesh=_MESH,
    scratch_types=[
        pltpu.VMEM((RPW_MAX, IDXW), jnp.int32),
        pltpu.VMEM((RPW_MAX, IDXW), jnp.int32),
        pltpu.VMEM((NB, IDXW, DOUT), jnp.float32),
        pltpu.VMEM((IDXW,), jnp.float32),
        pltpu.VMEM((ZROWS, DOUT), jnp.float32),
        pltpu.VMEM((N_SLAB,), jnp.float32),
        pltpu.VMEM_SHARED((N_PAD, DOUT), jnp.float32),
        pltpu.VMEM_SHARED((N_PAD,), jnp.float32),
        pltpu.SemaphoreType.DMA,
        pltpu.SemaphoreType.DMA,
        pltpu.SemaphoreType.DMA,
    ],
    compiler_params=_SC_PARAMS,
)


def _tc1_body(x_ref, w1t_ref, b1_ref, w2t_ref, b2_ref, xw1_ref, d_ref):
    x = x_ref[...]
    xw1_ref[...] = (
        jnp.dot(x, w1t_ref[...], preferred_element_type=jnp.float32)
        + b1_ref[...])
    d_ref[...] = (
        jnp.dot(x, w2t_ref[...], preferred_element_type=jnp.float32)
        + b2_ref[...])


_tc1 = pl.pallas_call(
    _tc1_body,
    out_shape=(
        jax.ShapeDtypeStruct((N, DOUT), jnp.float32),
        jax.ShapeDtypeStruct((N, DOUT), jnp.float32),
    ),
)


def _tc2_body(p0_ref, p1_ref, w2bt_ref, y_ref):
    xe = p0_ref[...] + p1_ref[...]
    y_ref[...] = jnp.dot(xe, w2bt_ref[...],
                         preferred_element_type=jnp.float32)


_tc2 = pl.pallas_call(
    _tc2_body,
    out_shape=jax.ShapeDtypeStruct((M_PAD, DOUT), jnp.float32),
)


def _tc3_body(d_ref, s0_ref, s1_ref, c_ref, x0_ref, wt_ref, wb_ref, out_ref):
    cnt = (c_ref[0, :] + c_ref[1, :])[:, None]
    xv = cnt * d_ref[...] + s0_ref[...] + s1_ref[...]
    xn = 0.5 * xv + 0.5 * x0_ref[...]
    out_ref[...] = (
        jnp.dot(xn, wt_ref[...], preferred_element_type=jnp.float32)
        + wb_ref[...])


_tc3 = pl.pallas_call(
    _tc3_body,
    out_shape=jax.ShapeDtypeStruct((N, DOUT), jnp.float32),
)


def kernel(X, vertex, edges, atts, X0, W1_w, W1_b, W2_w, W2_b, W_w, W_b):
    vg2d = jnp.concatenate(
        [vertex, jnp.zeros((PAD,), jnp.int32)]).reshape(ROWS, IDXW)
    vs2d = jnp.concatenate(
        [vertex, jnp.full((PAD,), N, jnp.int32)]).reshape(ROWS, IDXW)
    e2d = jnp.concatenate(
        [edges, jnp.full((PAD,), M, jnp.int32)]).reshape(ROWS, IDXW)
    a2d = jnp.concatenate(
        [atts.reshape(NNZ), jnp.zeros((PAD,), jnp.float32)]).reshape(ROWS, IDXW)

    w1t = W1_w.T
    w2at = W2_w[:, :DIN].T
    w2bt = W2_w[:, DIN:].T
    wt = W_w.T
    b1 = W1_b.reshape(1, DOUT)
    b2 = W2_b.reshape(1, DOUT)
    wb = W_b.reshape(1, DOUT)

    xw1, d = _tc1(X, w1t, b1, w2at, b2)
    xe_parts = _sc_a(xw1, vg2d, e2d, a2d)
    y = _tc2(xe_parts[0], xe_parts[1], w2bt)
    s_parts, cnt_parts = _sc_b(y, vs2d, e2d)
    out = _tc3(d, s_parts[0, :N], s_parts[1, :N], cnt_parts[:, :N],
               X0, wt, wb)
    return out


# speedup vs baseline: 1.7483x; 1.5176x over previous
import jax
import jax.numpy as jnp
from jax import lax
from jax.experimental import pallas as pl
from jax.experimental.pallas import tpu as pltpu
from jax.experimental.pallas import tpu_sc as plsc

N = 10000
M = 20000
NNZ = 320000
DIN = 128
DOUT = 32

NC = 2
NS = 16
NW = NC * NS
L = 16

IDXW = 128
RPW = 80
NB = 8
RPW0 = 80
RPW1 = 2 * RPW - RPW0
RPW_MAX = max(RPW0, RPW1)
ROWS = NW * RPW
NNZ_PAD = ROWS * IDXW
PAD = NNZ_PAD - NNZ

M_PAD = 20480
M_SLAB = M_PAD // NS
N_PAD = 10240
N_SLAB = N_PAD // NS
ZROWS = 160
TBL_A = 10240
TBL_SLAB_A = N // NS
TBL_SLAB_B = M_PAD // NS

_MESH = plsc.VectorSubcoreMesh(core_axis_name="c", subcore_axis_name="s")
_SC_PARAMS = pltpu.CompilerParams(use_tc_tiling_on_sc=False)


def _splat(vec, j):
    idx = jnp.full((L, 1), j, dtype=jnp.int32)
    dn = lax.GatherDimensionNumbers(
        offset_dims=(), collapsed_slice_dims=(0,), start_index_map=(0,))
    return lax.gather(vec, idx, dn, (1,),
                      mode=lax.GatherScatterMode.PROMISE_IN_BOUNDS)


def _stage(src2d, dst, cid, sid):
    @pl.when(cid == 0)
    def _():
        pltpu.sync_copy(src2d.at[pl.ds(sid * RPW0, RPW0)],
                        dst.at[pl.ds(0, RPW0)])

    @pl.when(cid == 1)
    def _():
        pltpu.sync_copy(src2d.at[pl.ds(NS * RPW0 + sid * RPW1, RPW1)],
                        dst.at[pl.ds(0, RPW1)])


def _zero_rows(zbuf, nrows):
    z = jnp.zeros((L,), jnp.float32)

    def body(i, _):
        zbuf[i, 0:16] = z
        zbuf[i, 16:32] = z
        return 0

    lax.fori_loop(0, nrows, body, 0, unroll=False)


def _sc_a_body(xw1, v2d, e2d, a2d, xe_out,
               vidx, eidx, attv, rows, zbuf, tbl, xe_acc, gsem, ssem, asem):
    cid = lax.axis_index("c")
    sid = lax.axis_index("s")

    with jax.named_scope("init"):
        pltpu.sync_copy(xw1.at[pl.ds(sid * TBL_SLAB_A, TBL_SLAB_A)],
                        tbl.at[pl.ds(sid * TBL_SLAB_A, TBL_SLAB_A)])
        _zero_rows(zbuf, ZROWS)
        for k in range(M_SLAB // ZROWS):
            pltpu.sync_copy(zbuf, xe_acc.at[pl.ds(sid * M_SLAB + k * ZROWS,
                                                  ZROWS)])
        _stage(v2d, vidx, cid, sid)
        _stage(e2d, eidx, cid, sid)
        plsc.subcore_barrier()

    nsteps = jnp.where(cid == 0, RPW0 // NB, RPW1 // NB)
    abase = jnp.where(cid == 0, sid * RPW0, NS * RPW0 + sid * RPW1)

    def scale_rows(j, b):
        def blk_body(blk, _):
            ab = attv[j, pl.ds(blk * L, L)]
            for r in range(L):
                s = _splat(ab, r)
                rr = blk * L + r
                rows[b, rr, 0:16] = rows[b, rr, 0:16] * s
                rows[b, rr, 16:32] = rows[b, rr, 16:32] * s
            return 0

        lax.fori_loop(0, IDXW // L, blk_body, 0, unroll=False)

    def body(step, _):
        a_cp = pltpu.async_copy(a2d.at[pl.ds(abase + step * NB, NB)],
                                attv, asem)
        g = []
        for b in range(NB):
            j = step * NB + b
            g.append(pltpu.async_copy(tbl.at[vidx.at[j]], rows.at[b], gsem))
        a_cp.wait()
        sc = []
        for b in range(NB):
            j = step * NB + b
            g[b].wait()
            scale_rows(b, b)
            sc.append(pltpu.async_copy(
                rows.at[b], xe_acc.at[eidx.at[j]], ssem, add=True))
        for d in sc:
            d.wait()
        return 0

    with jax.named_scope("gather_scatter"):
        lax.fori_loop(0, nsteps, body, 0, unroll=False)
        plsc.subcore_barrier()

    with jax.named_scope("writeback"):
        pltpu.sync_copy(xe_acc.at[pl.ds(sid * M_SLAB, M_SLAB)],
                        xe_out.at[cid, pl.ds(sid * M_SLAB, M_SLAB)])


_sc_a = pl.kernel(
    _sc_a_body,
    out_type=jax.ShapeDtypeStruct((NC, M_PAD, DOUT), jnp.float32),
    mesh=_MESH,
    scratch_types=[
        pltpu.VMEM((RPW_MAX, IDXW), jnp.int32),
        pltpu.VMEM((RPW_MAX, IDXW), jnp.int32),
        pltpu.VMEM((NB, IDXW), jnp.float32),
        pltpu.VMEM((NB, IDXW, DOUT), jnp.float32),
        pltpu.VMEM((ZROWS, DOUT), jnp.float32),
        pltpu.VMEM_SHARED((TBL_A, DOUT), jnp.float32),
        pltpu.VMEM_SHARED((M_PAD, DOUT), jnp.float32),
        pltpu.SemaphoreType.DMA,
        pltpu.SemaphoreType.DMA,
        pltpu.SemaphoreType.DMA,
    ],
    compiler_params=_SC_PARAMS,
)


def _sc_b_body(y, v2d, e2d, s_out, cnt_out,
               vidx, eidx, rows, ones, zbuf, zcnt, tbl, s_acc, cnt_acc,
               gsem, ssem, csem):
    cid = lax.axis_index("c")
    sid = lax.axis_index("s")

    pltpu.sync_copy(y.at[pl.ds(sid * TBL_SLAB_B, TBL_SLAB_B)],
                    tbl.at[pl.ds(sid * TBL_SLAB_B, TBL_SLAB_B)])
    _zero_rows(zbuf, ZROWS)
    one = jnp.full((L,), 1.0, jnp.float32)
    zv = jnp.zeros((L,), jnp.float32)
    for i in range(IDXW // L):
        ones[pl.ds(i * L, L)] = one
    for i in range(N_SLAB // L):
        zcnt[pl.ds(i * L, L)] = zv
    for k in range(N_SLAB // ZROWS):
        pltpu.sync_copy(zbuf, s_acc.at[pl.ds(sid * N_SLAB + k * ZROWS,
                                             ZROWS)])
    pltpu.sync_copy(zcnt, cnt_acc.at[pl.ds(sid * N_SLAB, N_SLAB)])
    _stage(v2d, vidx, cid, sid)
    _stage(e2d, eidx, cid, sid)
    plsc.subcore_barrier()

    nsteps = jnp.where(cid == 0, RPW0 // NB, RPW1 // NB)

    def body(step, _):
        g = []
        for b in range(NB):
            j = step * NB + b
            g.append(pltpu.async_copy(tbl.at[eidx.at[j]], rows.at[b], gsem))
        sc = []
        for b in range(NB):
            j = step * NB + b
            g[b].wait()
            sc.append(pltpu.async_copy(
                rows.at[b], s_acc.at[vidx.at[j]], ssem, add=True))
            sc.append(pltpu.async_copy(
                ones, cnt_acc.at[vidx.at[j]], csem, add=True))
        for d in sc:
            d.wait()
        return 0

    lax.fori_loop(0, nsteps, body, 0, unroll=False)
    plsc.subcore_barrier()

    pltpu.sync_copy(s_acc.at[pl.ds(sid * N_SLAB, N_SLAB)],
                    s_out.at[cid, pl.ds(sid * N_SLAB, N_SLAB)])
    pltpu.sync_copy(cnt_acc.at[pl.ds(sid * N_SLAB, N_SLAB)],
                    cnt_out.at[cid, pl.ds(sid * N_SLAB, N_SLAB)])


_sc_b = pl.kernel(
    _sc_b_body,
    out_type=(
        jax.ShapeDtypeStruct((NC, N_PAD, DOUT), jnp.float32),
        jax.ShapeDtypeStruct((NC, N_PAD), jnp.float32),
    ),
    mesh=_MESH,
    scratch_types=[
        pltpu.VMEM((RPW_MAX, IDXW), jnp.int32),
        pltpu.VMEM((RPW_MAX, IDXW), jnp.int32),
        pltpu.VMEM((NB, IDXW, DOUT), jnp.float32),
        pltpu.VMEM((IDXW,), jnp.float32),
        pltpu.VMEM((ZROWS, DOUT), jnp.float32),
        pltpu.VMEM((N_SLAB,), jnp.float32),
        pltpu.VMEM_SHARED((M_PAD, DOUT), jnp.float32),
        pltpu.VMEM_SHARED((N_PAD, DOUT), jnp.float32),
        pltpu.VMEM_SHARED((N_PAD,), jnp.float32),
        pltpu.SemaphoreType.DMA,
        pltpu.SemaphoreType.DMA,
        pltpu.SemaphoreType.DMA,
    ],
    compiler_params=_SC_PARAMS,
)


def _tc1_body(x_ref, w1t_ref, b1_ref, w2t_ref, b2_ref, xw1_ref, d_ref):
    x = x_ref[...]
    xw1_ref[...] = (
        jnp.dot(x, w1t_ref[...], preferred_element_type=jnp.float32)
        + b1_ref[...])
    d_ref[...] = (
        jnp.dot(x, w2t_ref[...], preferred_element_type=jnp.float32)
        + b2_ref[...])


_tc1 = pl.pallas_call(
    _tc1_body,
    out_shape=(
        jax.ShapeDtypeStruct((N, DOUT), jnp.float32),
        jax.ShapeDtypeStruct((N, DOUT), jnp.float32),
    ),
)


def _tc2_body(p0_ref, p1_ref, w2bt_ref, y_ref):
    xe = p0_ref[...] + p1_ref[...]
    y_ref[...] = jnp.dot(xe, w2bt_ref[...],
                         preferred_element_type=jnp.float32)


_tc2 = pl.pallas_call(
    _tc2_body,
    out_shape=jax.ShapeDtypeStruct((M_PAD, DOUT), jnp.float32),
)


def _tc3_body(d_ref, s0_ref, s1_ref, c_ref, x0_ref, wt_ref, wb_ref, out_ref):
    cnt = (c_ref[0, :] + c_ref[1, :])[:, None]
    xv = cnt * d_ref[...] + s0_ref[...] + s1_ref[...]
    xn = 0.5 * xv + 0.5 * x0_ref[...]
    out_ref[...] = (
        jnp.dot(xn, wt_ref[...], preferred_element_type=jnp.float32)
        + wb_ref[...])


_tc3 = pl.pallas_call(
    _tc3_body,
    out_shape=jax.ShapeDtypeStruct((N, DOUT), jnp.float32),
)


def kernel(X, vertex, edges, atts, X0, W1_w, W1_b, W2_w, W2_b, W_w, W_b):
    vg2d = jnp.concatenate(
        [vertex, jnp.zeros((PAD,), jnp.int32)]).reshape(ROWS, IDXW)
    vs2d = jnp.concatenate(
        [vertex, jnp.full((PAD,), N, jnp.int32)]).reshape(ROWS, IDXW)
    e2d = jnp.concatenate(
        [edges, jnp.full((PAD,), M, jnp.int32)]).reshape(ROWS, IDXW)
    a2d = jnp.concatenate(
        [atts.reshape(NNZ), jnp.zeros((PAD,), jnp.float32)]).reshape(ROWS, IDXW)

    w1t = W1_w.T
    w2at = W2_w[:, :DIN].T
    w2bt = W2_w[:, DIN:].T
    wt = W_w.T
    b1 = W1_b.reshape(1, DOUT)
    b2 = W2_b.reshape(1, DOUT)
    wb = W_b.reshape(1, DOUT)

    xw1, d = _tc1(X, w1t, b1, w2at, b2)
    xe_parts = _sc_a(xw1, vg2d, e2d, a2d)
    y = _tc2(xe_parts[0], xe_parts[1], w2bt)
    s_parts, cnt_parts = _sc_b(y, vs2d, e2d)
    out = _tc3(d, s_parts[0, :N], s_parts[1, :N], cnt_parts[:, :N],
               X0, wt, wb)
    return out


# speedup vs baseline: 2.0610x; 1.1789x over previous
import jax
import jax.numpy as jnp
from jax import lax
from jax.experimental import pallas as pl
from jax.experimental.pallas import tpu as pltpu
from jax.experimental.pallas import tpu_sc as plsc

N = 10000
M = 20000
NNZ = 320000
DIN = 128
DOUT = 32

NC = 2
NS = 16
NW = NC * NS
L = 16

IDXW = 128
RPW = 80
NB = 8
RPW0 = 80
RPW1 = 2 * RPW - RPW0
RPW_MAX = max(RPW0, RPW1)
ROWS = NW * RPW
NNZ_PAD = ROWS * IDXW
PAD = NNZ_PAD - NNZ

M_PAD = 20480
M_SLAB = M_PAD // NS
N_PAD = 10240
N_SLAB = N_PAD // NS
ZROWS = 160
TBL_A = 10240
TBL_SLAB_A = N // NS
TBL_SLAB_B = M_PAD // NS

_MESH = plsc.VectorSubcoreMesh(core_axis_name="c", subcore_axis_name="s")
_SC_PARAMS = pltpu.CompilerParams(use_tc_tiling_on_sc=False)


def _splat(vec, j):
    idx = jnp.full((L, 1), j, dtype=jnp.int32)
    dn = lax.GatherDimensionNumbers(
        offset_dims=(), collapsed_slice_dims=(0,), start_index_map=(0,))
    return lax.gather(vec, idx, dn, (1,),
                      mode=lax.GatherScatterMode.PROMISE_IN_BOUNDS)


def _stage(src2d, dst, cid, sid):
    @pl.when(cid == 0)
    def _():
        pltpu.sync_copy(src2d.at[pl.ds(sid * RPW0, RPW0)],
                        dst.at[pl.ds(0, RPW0)])

    @pl.when(cid == 1)
    def _():
        pltpu.sync_copy(src2d.at[pl.ds(NS * RPW0 + sid * RPW1, RPW1)],
                        dst.at[pl.ds(0, RPW1)])


def _zero_rows(zbuf, nrows):
    z = jnp.zeros((L,), jnp.float32)

    def body(i, _):
        zbuf[i, 0:16] = z
        zbuf[i, 16:32] = z
        return 0

    lax.fori_loop(0, nrows, body, 0, unroll=False)


def _sc_a_body(xw1, v2d, e2d, a2d, xe_out,
               vidx, eidx, attv, rows, zbuf, tbl, xe_acc, gsem, ssem, asem):
    cid = lax.axis_index("c")
    sid = lax.axis_index("s")

    with jax.named_scope("init"):
        pltpu.sync_copy(xw1.at[pl.ds(sid * TBL_SLAB_A, TBL_SLAB_A)],
                        tbl.at[pl.ds(sid * TBL_SLAB_A, TBL_SLAB_A)])
        _zero_rows(zbuf, ZROWS)
        for k in range(M_SLAB // ZROWS):
            pltpu.sync_copy(zbuf, xe_acc.at[pl.ds(sid * M_SLAB + k * ZROWS,
                                                  ZROWS)])
        _stage(v2d, vidx, cid, sid)
        _stage(e2d, eidx, cid, sid)
        plsc.subcore_barrier()

    nsteps = jnp.where(cid == 0, RPW0 // NB, RPW1 // NB)
    abase = jnp.where(cid == 0, sid * RPW0, NS * RPW0 + sid * RPW1)

    def scale_rows(j, b):
        def blk_body(blk, _):
            ab = attv[j, pl.ds(blk * L, L)]
            for r in range(L):
                s = _splat(ab, r)
                rr = blk * L + r
                rows[b, rr, 0:16] = rows[b, rr, 0:16] * s
                rows[b, rr, 16:32] = rows[b, rr, 16:32] * s
            return 0

        lax.fori_loop(0, IDXW // L, blk_body, 0, unroll=False)

    def body(step, _):
        a_cp = pltpu.async_copy(a2d.at[pl.ds(abase + step * NB, NB)],
                                attv, asem)
        g = []
        for b in range(NB):
            j = step * NB + b
            g.append(pltpu.async_copy(tbl.at[vidx.at[j]], rows.at[b], gsem))
        a_cp.wait()
        sc = []
        for b in range(NB):
            j = step * NB + b
            g[b].wait()
            scale_rows(b, b)
            sc.append(pltpu.async_copy(
                rows.at[b], xe_acc.at[eidx.at[j]], ssem, add=True))
        for d in sc:
            d.wait()
        return 0

    with jax.named_scope("gather_scatter"):
        lax.fori_loop(0, nsteps, body, 0, unroll=False)
        plsc.subcore_barrier()

    with jax.named_scope("writeback"):
        pltpu.sync_copy(xe_acc.at[pl.ds(sid * M_SLAB, M_SLAB)],
                        xe_out.at[cid, pl.ds(sid * M_SLAB, M_SLAB)])


_sc_a = pl.kernel(
    _sc_a_body,
    out_type=jax.ShapeDtypeStruct((NC, M_PAD, DOUT), jnp.float32),
    mesh=_MESH,
    scratch_types=[
        pltpu.VMEM((RPW_MAX, IDXW), jnp.int32),
        pltpu.VMEM((RPW_MAX, IDXW), jnp.int32),
        pltpu.VMEM((NB, IDXW), jnp.float32),
        pltpu.VMEM((NB, IDXW, DOUT), jnp.float32),
        pltpu.VMEM((ZROWS, DOUT), jnp.float32),
        pltpu.VMEM_SHARED((TBL_A, DOUT), jnp.float32),
        pltpu.VMEM_SHARED((M_PAD, DOUT), jnp.float32),
        pltpu.SemaphoreType.DMA,
        pltpu.SemaphoreType.DMA,
        pltpu.SemaphoreType.DMA,
    ],
    compiler_params=_SC_PARAMS,
)


def _sc_b_body(xe, v2d, e2d, s_out, cnt_out,
               vidx, eidx, rows, ones, zbuf, zcnt, zidx, tbl, s_acc,
               cnt_acc, gsem, ssem, csem):
    cid = lax.axis_index("c")
    sid = lax.axis_index("s")

    pltpu.sync_copy(xe.at[0, pl.ds(sid * TBL_SLAB_B, TBL_SLAB_B)],
                    tbl.at[pl.ds(sid * TBL_SLAB_B, TBL_SLAB_B)])
    iota = lax.iota(jnp.int32, L)
    for k in range(TBL_SLAB_B // ZROWS):
        off = sid * TBL_SLAB_B + k * ZROWS
        for c in range(ZROWS // L):
            zidx[pl.ds(c * L, L)] = iota + (off + c * L)
        pltpu.sync_copy(xe.at[1, pl.ds(off, ZROWS)], zbuf)
        pltpu.sync_copy(zbuf, tbl.at[zidx], add=True)
    _zero_rows(zbuf, ZROWS)
    one = jnp.full((L,), 1.0, jnp.float32)
    zv = jnp.zeros((L,), jnp.float32)
    for i in range(IDXW // L):
        ones[pl.ds(i * L, L)] = one
    for i in range(N_SLAB // L):
        zcnt[pl.ds(i * L, L)] = zv
    for k in range(N_SLAB // ZROWS):
        pltpu.sync_copy(zbuf, s_acc.at[pl.ds(sid * N_SLAB + k * ZROWS,
                                             ZROWS)])
    pltpu.sync_copy(zcnt, cnt_acc.at[pl.ds(sid * N_SLAB, N_SLAB)])
    _stage(v2d, vidx, cid, sid)
    _stage(e2d, eidx, cid, sid)
    plsc.subcore_barrier()

    nsteps = jnp.where(cid == 0, RPW0 // NB, RPW1 // NB)

    def body(step, _):
        g = []
        for b in range(NB):
            j = step * NB + b
            g.append(pltpu.async_copy(tbl.at[eidx.at[j]], rows.at[b], gsem))
        sc = []
        for b in range(NB):
            j = step * NB + b
            g[b].wait()
            sc.append(pltpu.async_copy(
                rows.at[b], s_acc.at[vidx.at[j]], ssem, add=True))
            sc.append(pltpu.async_copy(
                ones, cnt_acc.at[vidx.at[j]], csem, add=True))
        for d in sc:
            d.wait()
        return 0

    lax.fori_loop(0, nsteps, body, 0, unroll=False)
    plsc.subcore_barrier()

    pltpu.sync_copy(s_acc.at[pl.ds(sid * N_SLAB, N_SLAB)],
                    s_out.at[cid, pl.ds(sid * N_SLAB, N_SLAB)])
    pltpu.sync_copy(cnt_acc.at[pl.ds(sid * N_SLAB, N_SLAB)],
                    cnt_out.at[cid, pl.ds(sid * N_SLAB, N_SLAB)])


_sc_b = pl.kernel(
    _sc_b_body,
    out_type=(
        jax.ShapeDtypeStruct((NC, N_PAD, DOUT), jnp.float32),
        jax.ShapeDtypeStruct((NC, N_PAD), jnp.float32),
    ),
    mesh=_MESH,
    scratch_types=[
        pltpu.VMEM((RPW_MAX, IDXW), jnp.int32),
        pltpu.VMEM((RPW_MAX, IDXW), jnp.int32),
        pltpu.VMEM((NB, IDXW, DOUT), jnp.float32),
        pltpu.VMEM((IDXW,), jnp.float32),
        pltpu.VMEM((ZROWS, DOUT), jnp.float32),
        pltpu.VMEM((N_SLAB,), jnp.float32),
        pltpu.VMEM((ZROWS,), jnp.int32),
        pltpu.VMEM_SHARED((M_PAD, DOUT), jnp.float32),
        pltpu.VMEM_SHARED((N_PAD, DOUT), jnp.float32),
        pltpu.VMEM_SHARED((N_PAD,), jnp.float32),
        pltpu.SemaphoreType.DMA,
        pltpu.SemaphoreType.DMA,
        pltpu.SemaphoreType.DMA,
    ],
    compiler_params=_SC_PARAMS,
)


def _tc1_body(x_ref, w1t_ref, b1_ref, w2t_ref, b2_ref, xw1_ref, d_ref):
    x = x_ref[...]
    xw1_ref[...] = (
        jnp.dot(x, w1t_ref[...], preferred_element_type=jnp.float32)
        + b1_ref[...])
    d_ref[...] = (
        jnp.dot(x, w2t_ref[...], preferred_element_type=jnp.float32)
        + b2_ref[...])


_tc1 = pl.pallas_call(
    _tc1_body,
    out_shape=(
        jax.ShapeDtypeStruct((N, DOUT), jnp.float32),
        jax.ShapeDtypeStruct((N, DOUT), jnp.float32),
    ),
)


def _tc3_body(d_ref, s0_ref, s1_ref, c_ref, x0_ref, w2bt_ref, wt_ref,
              wb_ref, out_ref):
    cnt = (c_ref[0, :] + c_ref[1, :])[:, None]
    s = s0_ref[...] + s1_ref[...]
    xv = cnt * d_ref[...] + jnp.dot(s, w2bt_ref[...],
                                    preferred_element_type=jnp.float32)
    xn = 0.5 * xv + 0.5 * x0_ref[...]
    out_ref[...] = (
        jnp.dot(xn, wt_ref[...], preferred_element_type=jnp.float32)
        + wb_ref[...])


_tc3 = pl.pallas_call(
    _tc3_body,
    out_shape=jax.ShapeDtypeStruct((N, DOUT), jnp.float32),
)


def kernel(X, vertex, edges, atts, X0, W1_w, W1_b, W2_w, W2_b, W_w, W_b):
    vg2d = jnp.concatenate(
        [vertex, jnp.zeros((PAD,), jnp.int32)]).reshape(ROWS, IDXW)
    vs2d = jnp.concatenate(
        [vertex, jnp.full((PAD,), N, jnp.int32)]).reshape(ROWS, IDXW)
    e2d = jnp.concatenate(
        [edges, jnp.full((PAD,), M, jnp.int32)]).reshape(ROWS, IDXW)
    a2d = jnp.concatenate(
        [atts.reshape(NNZ), jnp.zeros((PAD,), jnp.float32)]).reshape(ROWS, IDXW)

    w1t = W1_w.T
    w2at = W2_w[:, :DIN].T
    w2bt = W2_w[:, DIN:].T
    wt = W_w.T
    b1 = W1_b.reshape(1, DOUT)
    b2 = W2_b.reshape(1, DOUT)
    wb = W_b.reshape(1, DOUT)

    xw1, d = _tc1(X, w1t, b1, w2at, b2)
    xe_parts = _sc_a(xw1, vg2d, e2d, a2d)
    s_parts, cnt_parts = _sc_b(xe_parts, vs2d, e2d)
    out = _tc3(d, s_parts[0, :N], s_parts[1, :N], cnt_parts[:, :N],
               X0, w2bt, wt, wb)
    return out
